# baseline (speedup 1.0000x reference)
import functools
import math

import jax
import jax.numpy as jnp
from jax import lax
from jax.experimental import pallas as pl
from jax.experimental.pallas import tpu as pltpu, tpu_sc as plsc

N, D, E = 10000, 128, 320000
BN_EPS = 1e-5

NS = 16
EPT = E // NS
CHUNK = 80
NCHUNK = EPT // CHUNK
NBUF = 4
NROUND = -(-(NCHUNK + 2) // NBUF)
STRIPE = 632
LAST_STRIPE = N - (NS - 1) * STRIPE


def _make_sc_aggregate():
    mesh = plsc.VectorSubcoreMesh(core_axis_name="c", subcore_axis_name="s")

    @functools.partial(
        pl.kernel,
        mesh=mesh,
        out_type=[
            jax.ShapeDtypeStruct((N, D), jnp.float32),
            jax.ShapeDtypeStruct((N, D), jnp.float32),
        ],
        scratch_types=(
            [pltpu.VMEM_SHARED((N, D), jnp.float32)]
            + [pltpu.VMEM((CHUNK, D), jnp.float32)] * NBUF
            + [pltpu.VMEM((CHUNK,), jnp.int32)] * (2 * NBUF)
            + [pltpu.SemaphoreType.DMA] * (3 * NBUF)
        ),
    )
    def sc_aggr(mi_flat, eflat, aggr0, aggr1, accum, *bufs):
        rows = bufs[:NBUF]
        sidx = bufs[NBUF:2 * NBUF]
        didx = bufs[2 * NBUF:3 * NBUF]
        gsem = bufs[3 * NBUF:4 * NBUF]
        ssem = bufs[4 * NBUF:5 * NBUF]
        isem = bufs[5 * NBUF:6 * NBUF]
        cid = lax.axis_index("c")
        sid = lax.axis_index("s")
        r0 = pl.multiple_of(sid * STRIPE, 8)

        def stripe_copy(src_ref, dst_ref):
            @pl.when(sid < NS - 1)
            def _():
                pltpu.sync_copy(src_ref.at[pl.ds(r0, STRIPE)],
                                dst_ref.at[pl.ds(r0, STRIPE)])

            @pl.when(sid == NS - 1)
            def _():
                pltpu.sync_copy(src_ref.at[pl.ds((NS - 1) * STRIPE, LAST_STRIPE)],
                                dst_ref.at[pl.ds((NS - 1) * STRIPE, LAST_STRIPE)])

        def zbody(r, carry):
            for c in range(D // 16):
                rows[0][r, pl.ds(c * 16, 16)] = jnp.zeros((16,), jnp.float32)
            return carry

        lax.fori_loop(0, CHUNK, zbody, 0)

        @pl.when(sid < NS - 1)
        def _():
            for i in range(STRIPE // CHUNK):
                pltpu.sync_copy(rows[0],
                                accum.at[pl.ds(r0 + i * CHUNK, CHUNK)])
            rem = STRIPE % CHUNK
            pltpu.sync_copy(
                rows[0].at[pl.ds(0, rem)],
                accum.at[pl.ds(r0 + STRIPE - rem, rem)])

        @pl.when(sid == NS - 1)
        def _():
            base = (NS - 1) * STRIPE
            for i in range(LAST_STRIPE // CHUNK):
                pltpu.sync_copy(rows[0],
                                accum.at[pl.ds(base + i * CHUNK, CHUNK)])
            rem = LAST_STRIPE % CHUNK
            pltpu.sync_copy(
                rows[0].at[pl.ds(0, rem)],
                accum.at[pl.ds(base + LAST_STRIPE - rem, rem)])

        plsc.subcore_barrier()

        ebase = pl.multiple_of(sid * EPT, 8)

        def run(ci, roff):
            sbase = 2 * ci * E
            dbase = 2 * ci * E + E

            def wait_scatter(b):
                pltpu.make_async_copy(rows[b], accum.at[didx[b]],
                                      ssem[b]).wait()

            def visit(j, u):
                b2 = u
                b1 = (u + 3) % NBUF
                b0 = (u + 2) % NBUF

                @pl.when(j >= 2)
                def _():
                    wait_scatter(b2)

                @pl.when(j + 2 < NCHUNK)
                def _():
                    off = pl.multiple_of(ebase + (j + 2) * CHUNK, 8)
                    pltpu.async_copy(eflat.at[pl.ds(sbase + off, CHUNK)],
                                     sidx[b2], isem[b2])
                    pltpu.async_copy(eflat.at[pl.ds(dbase + off, CHUNK)],
                                     didx[b2], isem[b2])

                @pl.when((j + 1 >= 0) & (j + 1 < NCHUNK))
                def _():
                    pltpu.make_async_copy(eflat.at[pl.ds(0, CHUNK)],
                                          sidx[b1], isem[b1]).wait()
                    pltpu.make_async_copy(eflat.at[pl.ds(0, CHUNK)],
                                          didx[b1], isem[b1]).wait()
                    for c in range(CHUNK // 16):
                        sl = pl.ds(c * 16, 16)
                        sidx[b1][sl] = sidx[b1][sl] + roff
                    pltpu.async_copy(mi_flat.at[sidx[b1]], rows[b1], gsem[b1])

                @pl.when((j >= 0) & (j < NCHUNK))
                def _():
                    pltpu.make_async_copy(mi_flat.at[sidx[b0]], rows[b0],
                                          gsem[b0]).wait()
                    pltpu.async_copy(rows[b0], accum.at[didx[b0]],
                                     ssem[b0], add=True)

            def round_body(g, carry):
                for u in range(NBUF):
                    visit(NBUF * g + u - 2, u)
                return carry

            lax.fori_loop(0, NROUND, round_body, 0)
            for k in range(NROUND * NBUF - 4, NCHUNK):
                wait_scatter(k % NBUF)

        @pl.when(cid == 0)
        def _():
            run(0, N)

        @pl.when(cid == 1)
        def _():
            run(1, 2 * N)

        plsc.subcore_barrier()

        @pl.when(cid == 0)
        def _():
            stripe_copy(accum, aggr0)

        @pl.when(cid == 1)
        def _():
            stripe_copy(accum, aggr1)

    return sc_aggr


_sc_aggregate = _make_sc_aggregate()

_BLK = 1000


_BN_C = 1.0 / math.sqrt(1.0 + BN_EPS)


def _mlp_block(h, w0, b0, g, be, w1, b1):
    h = jnp.dot(h, w0[...], preferred_element_type=jnp.float32) + b0[...]
    h = h * (g[...] * _BN_C) + be[...]
    h = jnp.maximum(h, 0.0)
    return jnp.dot(h, w1[...], preferred_element_type=jnp.float32) + b1[...]


def _mi_spec(k):
    return pl.BlockSpec((1, _BLK, D), lambda i, k=k: (k, i, 0))


_row_spec = pl.BlockSpec((_BLK, D), lambda i: (i, 0))
_mat_spec = pl.BlockSpec((D, D), lambda i: (0, 0))
_vec_spec = pl.BlockSpec((1, D), lambda i: (0, 0))
_MLP_SPECS = [_mat_spec, _vec_spec, _vec_spec, _vec_spec, _mat_spec, _vec_spec]


def _tc_comb_body(eps0, eps1, x0, x1, x2, a0, a1,
                  wl0, bl0, sl, bel, wl1, bl1,
                  w00, b00, s0, be0, w01, b01,
                  w10, b10, s1, be1, w11, b11, out):
    acc = _mlp_block(x0[0], wl0, bl0, sl, bel, wl1, bl1)
    acc = acc + _mlp_block((1.0 + eps0[0]) * x1[0] + a0[...],
                           w00, b00, s0, be0, w01, b01)
    acc = acc + _mlp_block((1.0 + eps1[0]) * x2[0] + a1[...],
                           w10, b10, s1, be1, w11, b11)
    out[...] = acc


def _tc_combine(eps0, eps1, multi_input, a0, a1, mats):
    smem_spec = pl.BlockSpec(memory_space=pltpu.SMEM)
    in_specs = ([smem_spec, smem_spec]
                + [_mi_spec(0), _mi_spec(1), _mi_spec(2), _row_spec, _row_spec]
                + _MLP_SPECS * 3)
    return pl.pallas_call(
        _tc_comb_body,
        grid=(N // _BLK,),
        in_specs=in_specs,
        out_specs=_row_spec,
        out_shape=jax.ShapeDtypeStruct((N, D), jnp.float32),
    )(eps0.reshape(1), eps1.reshape(1), multi_input, multi_input, multi_input,
      a0, a1, *mats)


def kernel(multi_input, edge_index_list, lin_W0, lin_b0, lin_g, lin_be,
           lin_W1, lin_b1, c0_W0, c0_b0, c0_g, c0_be, c0_W1, c0_b1, eps0,
           c1_W0, c1_b0, c1_g, c1_be, c1_W1, c1_b1, eps1):
    mi_flat = multi_input.reshape(3 * N, D)
    eflat = edge_index_list.reshape(4 * E)

    aggr0, aggr1 = _sc_aggregate(mi_flat, eflat)

    row = lambda v: v.reshape(1, D)
    mats = [lin_W0, row(lin_b0), row(lin_g), row(lin_be), lin_W1, row(lin_b1),
            c0_W0, row(c0_b0), row(c0_g), row(c0_be), c0_W1, row(c0_b1),
            c1_W0, row(c1_b0), row(c1_g), row(c1_be), c1_W1, row(c1_b1)]

    return _tc_combine(eps0, eps1, multi_input, aggr0, aggr1, mats)


# speedup vs baseline: 1.0175x; 1.0175x over previous
import functools
import math

import jax
import jax.numpy as jnp
from jax import lax
from jax.experimental import pallas as pl
from jax.experimental.pallas import tpu as pltpu, tpu_sc as plsc

N, D, E = 10000, 128, 320000
BN_EPS = 1e-5

NS = 16
EPT = E // NS
CHUNK = 80
NCHUNK = EPT // CHUNK
NBUF = 4
NROUND = -(-(NCHUNK + 2) // NBUF)
STRIPE = 632
LAST_STRIPE = N - (NS - 1) * STRIPE


def _make_sc_aggregate():
    mesh = plsc.VectorSubcoreMesh(core_axis_name="c", subcore_axis_name="s")

    @functools.partial(
        pl.kernel,
        mesh=mesh,
        out_type=[
            jax.ShapeDtypeStruct((N, D), jnp.float32),
            jax.ShapeDtypeStruct((N, D), jnp.float32),
        ],
        scratch_types=(
            [pltpu.VMEM_SHARED((N, D), jnp.float32)]
            + [pltpu.VMEM((CHUNK, D), jnp.float32)] * NBUF
            + [pltpu.VMEM((CHUNK,), jnp.int32)] * (2 * NBUF)
            + [pltpu.SemaphoreType.DMA] * (3 * NBUF)
        ),
    )
    def sc_aggr(mi_flat, eflat, aggr0, aggr1, accum, *bufs):
        rows = bufs[:NBUF]
        sidx = bufs[NBUF:2 * NBUF]
        didx = bufs[2 * NBUF:3 * NBUF]
        gsem = bufs[3 * NBUF:4 * NBUF]
        ssem = bufs[4 * NBUF:5 * NBUF]
        isem = bufs[5 * NBUF:6 * NBUF]
        cid = lax.axis_index("c")
        sid = lax.axis_index("s")
        r0 = pl.multiple_of(sid * STRIPE, 8)

        def stripe_copy(src_ref, dst_ref):
            @pl.when(sid < NS - 1)
            def _():
                pltpu.sync_copy(src_ref.at[pl.ds(r0, STRIPE)],
                                dst_ref.at[pl.ds(r0, STRIPE)])

            @pl.when(sid == NS - 1)
            def _():
                pltpu.sync_copy(src_ref.at[pl.ds((NS - 1) * STRIPE, LAST_STRIPE)],
                                dst_ref.at[pl.ds((NS - 1) * STRIPE, LAST_STRIPE)])

        def zbody(r, carry):
            for c in range(D // 16):
                rows[0][r, pl.ds(c * 16, 16)] = jnp.zeros((16,), jnp.float32)
            return carry

        lax.fori_loop(0, CHUNK, zbody, 0)

        @pl.when(sid < NS - 1)
        def _():
            for i in range(STRIPE // CHUNK):
                pltpu.sync_copy(rows[0],
                                accum.at[pl.ds(r0 + i * CHUNK, CHUNK)])
            rem = STRIPE % CHUNK
            pltpu.sync_copy(
                rows[0].at[pl.ds(0, rem)],
                accum.at[pl.ds(r0 + STRIPE - rem, rem)])

        @pl.when(sid == NS - 1)
        def _():
            base = (NS - 1) * STRIPE
            for i in range(LAST_STRIPE // CHUNK):
                pltpu.sync_copy(rows[0],
                                accum.at[pl.ds(base + i * CHUNK, CHUNK)])
            rem = LAST_STRIPE % CHUNK
            pltpu.sync_copy(
                rows[0].at[pl.ds(0, rem)],
                accum.at[pl.ds(base + LAST_STRIPE - rem, rem)])

        plsc.subcore_barrier()

        ebase = pl.multiple_of(sid * EPT, 8)

        def run(ci, roff):
            sbase = 2 * ci * E
            dbase = 2 * ci * E + E

            def wait_scatter(b):
                pltpu.make_async_copy(rows[b], accum.at[didx[b]],
                                      ssem[b]).wait()

            def visit(j, u):
                b2 = u
                b1 = (u + 3) % NBUF
                b0 = (u + 2) % NBUF

                @pl.when(j >= 2)
                def _():
                    wait_scatter(b2)

                @pl.when(j + 2 < NCHUNK)
                def _():
                    off = pl.multiple_of(ebase + (j + 2) * CHUNK, 8)
                    pltpu.async_copy(eflat.at[pl.ds(sbase + off, CHUNK)],
                                     sidx[b2], isem[b2])
                    pltpu.async_copy(eflat.at[pl.ds(dbase + off, CHUNK)],
                                     didx[b2], isem[b2])

                @pl.when((j + 1 >= 0) & (j + 1 < NCHUNK))
                def _():
                    pltpu.make_async_copy(eflat.at[pl.ds(0, CHUNK)],
                                          sidx[b1], isem[b1]).wait()
                    pltpu.make_async_copy(eflat.at[pl.ds(0, CHUNK)],
                                          didx[b1], isem[b1]).wait()
                    for c in range(CHUNK // 16):
                        sl = pl.ds(c * 16, 16)
                        sidx[b1][sl] = sidx[b1][sl] + roff
                    pltpu.async_copy(mi_flat.at[sidx[b1]], rows[b1], gsem[b1])

                @pl.when((j >= 0) & (j < NCHUNK))
                def _():
                    pltpu.make_async_copy(mi_flat.at[sidx[b0]], rows[b0],
                                          gsem[b0]).wait()
                    pltpu.async_copy(rows[b0], accum.at[didx[b0]],
                                     ssem[b0], add=True)

            def round_body(g, carry):
                for u in range(NBUF):
                    visit(NBUF * g + u - 2, u)
                return carry

            lax.fori_loop(0, NROUND, round_body, 0)
            for k in range(NROUND * NBUF - 4, NCHUNK):
                wait_scatter(k % NBUF)

        @pl.when(cid == 0)
        def _():
            run(0, N)

        @pl.when(cid == 1)
        def _():
            run(1, 2 * N)

        plsc.subcore_barrier()

        @pl.when(cid == 0)
        def _():
            stripe_copy(accum, aggr0)

        @pl.when(cid == 1)
        def _():
            stripe_copy(accum, aggr1)

    return sc_aggr


_sc_aggregate = _make_sc_aggregate()

_BLK = 2000


_BN_C = 1.0 / math.sqrt(1.0 + BN_EPS)


def _mlp_block(h, w0, b0, g, be, w1, b1):
    h = jnp.dot(h, w0[...], preferred_element_type=jnp.float32) + b0[...]
    h = h * (g[...] * _BN_C) + be[...]
    h = jnp.maximum(h, 0.0)
    return jnp.dot(h, w1[...], preferred_element_type=jnp.float32) + b1[...]


def _mi_spec(k):
    return pl.BlockSpec((1, _BLK, D), lambda i, k=k: (k, i, 0))


_row_spec = pl.BlockSpec((_BLK, D), lambda i: (i, 0))
_mat_spec = pl.BlockSpec((D, D), lambda i: (0, 0))
_vec_spec = pl.BlockSpec((1, D), lambda i: (0, 0))
_MLP_SPECS = [_mat_spec, _vec_spec, _vec_spec, _vec_spec, _mat_spec, _vec_spec]


def _tc_comb_body(eps0, eps1, x0, x1, x2, a0, a1,
                  wl0, bl0, sl, bel, wl1, bl1,
                  w00, b00, s0, be0, w01, b01,
                  w10, b10, s1, be1, w11, b11, out):
    acc = _mlp_block(x0[0], wl0, bl0, sl, bel, wl1, bl1)
    acc = acc + _mlp_block((1.0 + eps0[0]) * x1[0] + a0[...],
                           w00, b00, s0, be0, w01, b01)
    acc = acc + _mlp_block((1.0 + eps1[0]) * x2[0] + a1[...],
                           w10, b10, s1, be1, w11, b11)
    out[...] = acc


def _tc_combine(eps0, eps1, multi_input, a0, a1, mats):
    smem_spec = pl.BlockSpec(memory_space=pltpu.SMEM)
    in_specs = ([smem_spec, smem_spec]
                + [_mi_spec(0), _mi_spec(1), _mi_spec(2), _row_spec, _row_spec]
                + _MLP_SPECS * 3)
    return pl.pallas_call(
        _tc_comb_body,
        grid=(N // _BLK,),
        in_specs=in_specs,
        out_specs=_row_spec,
        out_shape=jax.ShapeDtypeStruct((N, D), jnp.float32),
        compiler_params=pltpu.CompilerParams(
            dimension_semantics=("parallel",)),
    )(eps0.reshape(1), eps1.reshape(1), multi_input, multi_input, multi_input,
      a0, a1, *mats)


def kernel(multi_input, edge_index_list, lin_W0, lin_b0, lin_g, lin_be,
           lin_W1, lin_b1, c0_W0, c0_b0, c0_g, c0_be, c0_W1, c0_b1, eps0,
           c1_W0, c1_b0, c1_g, c1_be, c1_W1, c1_b1, eps1):
    mi_flat = multi_input.reshape(3 * N, D)
    eflat = edge_index_list.reshape(4 * E)

    aggr0, aggr1 = _sc_aggregate(mi_flat, eflat)

    row = lambda v: v.reshape(1, D)
    mats = [lin_W0, row(lin_b0), row(lin_g), row(lin_be), lin_W1, row(lin_b1),
            c0_W0, row(c0_b0), row(c0_g), row(c0_be), c0_W1, row(c0_b1),
            c1_W0, row(c1_b0), row(c1_g), row(c1_be), c1_W1, row(c1_b1)]

    return _tc_combine(eps0, eps1, multi_input, aggr0, aggr1, mats)
